# Initial kernel scaffold; baseline (speedup 1.0000x reference)
import jax
import jax.numpy as jnp
from jax.experimental import pallas as pl


def kernel(x_in, edge_index, edge_attr, params):
    raise NotImplementedError("write your pallas kernel here")


# speedup vs baseline: 1.0047x; 1.0047x over previous
import functools
import jax
import jax.numpy as jnp
from jax.experimental import pallas as pl
from jax.experimental.pallas import tpu as pltpu

_N = 10000
_E = 320000
_D = 128
_OD = 128
_ED = 16
_EB = 3200


def _full(shape):
    nd = len(shape)
    return pl.BlockSpec(shape, lambda i: (0,) * nd)


def _eblk(width):
    return pl.BlockSpec((_EB, width), lambda i: (i, 0))


def _ln(x, g, b, eps=1e-6):
    m = jnp.mean(x, axis=-1, keepdims=True)
    v = jnp.mean((x - m) * (x - m), axis=-1, keepdims=True)
    return (x - m) * jax.lax.rsqrt(v + eps) * g + b


def _node_pre_body(x_ref, dummy_ref, g0_ref, b0_ref, projw_ref, projb_ref,
                   gatew_ref, gateb_ref, w1d_ref, w1s_ref,
                   pd_ref, ps_ref, skip_ref, gate_ref):
    x = x_ref[...]
    invalid = x[:, 0:1] == -999.0
    x = jnp.where(invalid, dummy_ref[...], x)
    x = _ln(x, g0_ref[...], b0_ref[...])
    skip = x @ projw_ref[...] + projb_ref[...]
    gate = jax.nn.sigmoid(skip @ gatew_ref[...] + gateb_ref[...])
    skip_ref[...] = skip
    gate_ref[...] = gate
    pd_ref[...] = x @ w1d_ref[...]
    ps_ref[...] = x @ w1s_ref[...]


def _edge1_body(ea_ref, gd_ref, gs_ref,
                lng_ref, lnb_ref, w1_ref, b1_ref, w2_ref, b2_ref,
                w3_ref, b3_ref, cw1_ref, cb1_ref, cw2_ref, cb2_ref,
                e1w_ref, e1b_ref, e2w_ref, e2b_ref, e3w_ref, e3b_ref,
                eenc_ref, m1_ref):
    ea = ea_ref[...]
    h = _ln(ea, lng_ref[...], lnb_ref[...])
    h = jnp.maximum(h @ w1_ref[...] + b1_ref[...], 0.0)
    h = jnp.maximum(h @ w2_ref[...] + b2_ref[...], 0.0)
    eenc = h @ w3_ref[...] + b3_ref[...]
    ew = jnp.maximum(ea @ cw1_ref[...] + cb1_ref[...], 0.0)
    ew = jax.nn.sigmoid(ew @ cw2_ref[...] + cb2_ref[...])
    eenc = eenc * ew
    eenc_ref[...] = eenc
    h1 = jnp.maximum(gd_ref[...] + gs_ref[...] + eenc @ e1w_ref[...]
                     + e1b_ref[...], 0.0)
    h1 = jnp.maximum(h1 @ e2w_ref[...] + e2b_ref[...], 0.0)
    m1_ref[...] = h1 @ e3w_ref[...] + e3b_ref[...]


def _edge2_body(eenc_ref, gd_ref, gs_ref,
                wd_ref, ws_ref, we_ref, b1_ref, w2_ref, b2_ref,
                w3_ref, b3_ref, m2_ref):
    h = jnp.maximum(gd_ref[...] @ wd_ref[...] + gs_ref[...] @ ws_ref[...]
                    + eenc_ref[...] @ we_ref[...] + b1_ref[...], 0.0)
    h = jnp.maximum(h @ w2_ref[...] + b2_ref[...], 0.0)
    m2_ref[...] = h @ w3_ref[...] + b3_ref[...]


def _node_mid_body(s1_ref, cnt_ref, g_ref, b_ref, x1_ref):
    x1 = s1_ref[...] / jnp.maximum(cnt_ref[...], 1.0)
    x1 = _ln(x1, g_ref[...], b_ref[...])
    x1_ref[...] = jnp.where(x1 > 0, x1, 0.01 * x1)


def _node_fin_body(s2_ref, cnt_ref, efs_ref, skip_ref, gate_ref,
                   g2_ref, b2_ref, w1_ref, bb1_ref, w2_ref, bb2_ref,
                   w3_ref, bb3_ref, xfc_ref, probs_ref):
    cnt = jnp.maximum(cnt_ref[...], 1.0)
    x2 = _ln(s2_ref[...] / cnt, g2_ref[...], b2_ref[...])
    x2 = jnp.maximum(x2, 0.0)
    gate = gate_ref[...]
    xf = gate * skip_ref[...] + (1.0 - gate) * x2
    efm = efs_ref[...] / cnt
    xfc = jnp.concatenate([xf, efm], axis=1)
    xfc_ref[...] = xfc
    h = xfc @ w1_ref[...] + bb1_ref[...]
    h = jnp.where(h > 0, h, jnp.exp(jnp.minimum(h, 0.0)) - 1.0)
    h = h @ w2_ref[...] + bb2_ref[...]
    h = jnp.where(h > 0, h, jnp.exp(jnp.minimum(h, 0.0)) - 1.0)
    probs_ref[...] = h @ w3_ref[...] + bb3_ref[...]


def _f32(shape):
    return jax.ShapeDtypeStruct(shape, jnp.float32)


@jax.jit
def _run(x_in, edge_index, edge_attr, params):
    p = params
    x = x_in[0]
    ea = edge_attr[0]
    src = edge_index[0, 0]
    dst = edge_index[0, 1]
    r = lambda b: b.reshape(1, -1)

    c1w1 = p['c1_W1']
    w1d = c1w1[:, :_D].T
    w1s = c1w1[:, _D:2 * _D].T
    w1e = c1w1[:, 2 * _D:].T
    c2w1 = p['c2_W1']
    w2d = c2w1[:, :_OD // 2].T
    w2s = c2w1[:, _OD // 2:_OD].T
    w2e = c2w1[:, _OD:].T

    pd, ps, skip, gate = pl.pallas_call(
        _node_pre_body,
        out_shape=(_f32((_N, _OD // 2)), _f32((_N, _OD // 2)),
                   _f32((_N, _OD)), _f32((_N, _OD))),
        in_specs=[_full((_N, _D)), _full((1, _D)), _full((1, _D)),
                  _full((1, _D)), _full((_D, _OD)), _full((1, _OD)),
                  _full((_OD, _OD)), _full((1, _OD)),
                  _full((_D, _OD // 2)), _full((_D, _OD // 2))],
        out_specs=(_full((_N, _OD // 2)), _full((_N, _OD // 2)),
                   _full((_N, _OD)), _full((_N, _OD))),
        grid=(1,),
    )(x, r(p['dummy']), r(p['bn0_g']), r(p['bn0_b']),
      p['proj_W'].T, r(p['proj_b']), p['gate_W'].T, r(p['gate_b']),
      w1d, w1s)

    gd1 = jnp.take(pd, dst, axis=0)
    gs1 = jnp.take(ps, src, axis=0)

    nblk = _E // _EB
    eenc, m1 = pl.pallas_call(
        _edge1_body,
        out_shape=(_f32((_E, _OD)), _f32((_E, _OD // 2))),
        in_specs=[_eblk(_ED), _eblk(_OD // 2), _eblk(_OD // 2),
                  _full((1, _ED)), _full((1, _ED)),
                  _full((_ED, _OD)), _full((1, _OD)),
                  _full((_OD, 2 * _OD)), _full((1, 2 * _OD)),
                  _full((2 * _OD, _OD)), _full((1, _OD)),
                  _full((_ED, _ED)), _full((1, _ED)),
                  _full((_ED, 1)), _full((1, 1)),
                  _full((_OD, _OD // 2)), _full((1, _OD // 2)),
                  _full((_OD // 2, _OD // 2)), _full((1, _OD // 2)),
                  _full((_OD // 2, _OD // 2)), _full((1, _OD // 2))],
        out_specs=(_eblk(_OD), _eblk(_OD // 2)),
        grid=(nblk,),
    )(ea, gd1, gs1,
      r(p['ee_ln_g']), r(p['ee_ln_b']),
      p['ee_W1'].T, r(p['ee_b1']), p['ee_W2'].T, r(p['ee_b2']),
      p['ee_W3'].T, r(p['ee_b3']),
      p['ec_W1'].T, r(p['ec_b1']), p['ec_W2'].T, r(p['ec_b2']),
      w1e, r(p['c1_b1']), p['c1_W2'].T, r(p['c1_b2']),
      p['c1_W3'].T, r(p['c1_b3']))

    ones = jnp.ones((_E,), jnp.float32)
    cnt = jax.ops.segment_sum(ones, dst, num_segments=_N).reshape(_N, 1)
    s1 = jax.ops.segment_sum(m1, dst, num_segments=_N)

    x1 = pl.pallas_call(
        _node_mid_body,
        out_shape=_f32((_N, _OD // 2)),
        in_specs=[_full((_N, _OD // 2)), _full((_N, 1)),
                  _full((1, _OD // 2)), _full((1, _OD // 2))],
        out_specs=_full((_N, _OD // 2)),
        grid=(1,),
    )(s1, cnt, r(p['bn1_g']), r(p['bn1_b']))

    g2d = jnp.take(x1, dst, axis=0)
    g2s = jnp.take(x1, src, axis=0)

    m2 = pl.pallas_call(
        _edge2_body,
        out_shape=_f32((_E, _OD)),
        in_specs=[_eblk(_OD), _eblk(_OD // 2), _eblk(_OD // 2),
                  _full((_OD // 2, _OD)), _full((_OD // 2, _OD)),
                  _full((_OD, _OD)), _full((1, _OD)),
                  _full((_OD, _OD)), _full((1, _OD)),
                  _full((_OD, _OD)), _full((1, _OD))],
        out_specs=_eblk(_OD),
        grid=(nblk,),
    )(eenc, g2d, g2s,
      w2d, w2s, w2e, r(p['c2_b1']),
      p['c2_W2'].T, r(p['c2_b2']), p['c2_W3'].T, r(p['c2_b3']))

    s2 = jax.ops.segment_sum(m2, dst, num_segments=_N)
    efs = jax.ops.segment_sum(eenc, dst, num_segments=_N)

    xfc, probs = pl.pallas_call(
        _node_fin_body,
        out_shape=(_f32((_N, 2 * _OD)), _f32((_N, 1))),
        in_specs=[_full((_N, _OD)), _full((_N, 1)), _full((_N, _OD)),
                  _full((_N, _OD)), _full((_N, _OD)),
                  _full((1, _OD)), _full((1, _OD)),
                  _full((2 * _OD, _OD)), _full((1, _OD)),
                  _full((_OD, _OD // 2)), _full((1, _OD // 2)),
                  _full((_OD // 2, 1)), _full((1, 1))],
        out_specs=(_full((_N, 2 * _OD)), _full((_N, 1))),
        grid=(1,),
    )(s2, cnt, efs, skip, gate,
      r(p['bn2_g']), r(p['bn2_b']),
      p['np_W1'].T, r(p['np_b1']), p['np_W2'].T, r(p['np_b2']),
      p['np_W3'].T, r(p['np_b3']))

    return xfc, probs


def kernel(x_in, edge_index, edge_attr, params):
    return _run(x_in, edge_index, edge_attr, params)


# speedup vs baseline: 2.7381x; 2.7253x over previous
import dataclasses
import functools
import jax
import jax.numpy as jnp
from jax import lax
from jax.experimental import pallas as pl
from jax.experimental.pallas import tpu as pltpu
from jax.experimental.pallas import tpu_sc as plsc

_N = 10000
_E = 320000
_D = 128
_OD = 128
_ED = 16
_EB = 3200
_NC, _NS = 2, 16
_NW = _NC * _NS
_EPW = _E // _NW
_K = 80
_NCHUNK = _EPW // _K
_NP = 10240
_STRIPE = _NP // _NS

_sc_mesh = plsc.VectorSubcoreMesh(core_axis_name="c", subcore_axis_name="s")


def _full(shape):
    nd = len(shape)
    return pl.BlockSpec(shape, lambda i: (0,) * nd)


def _eblk(width):
    return pl.BlockSpec((_EB, width), lambda i: (i, 0))


def _f32(shape):
    return jax.ShapeDtypeStruct(shape, jnp.float32)


def _ln(x, g, b, eps=1e-6):
    m = jnp.mean(x, axis=-1, keepdims=True)
    v = jnp.mean((x - m) * (x - m), axis=-1, keepdims=True)
    return (x - m) * jax.lax.rsqrt(v + eps) * g + b


def _sc_gather_pair_body(ta_hbm, tb_hbm, ia_hbm, ib_hbm, oa_hbm, ob_hbm,
                         idxa, idxb, rowsa, rowsb, sema, semb):
    wid = lax.axis_index("s") * _NC + lax.axis_index("c")
    base = wid * _EPW

    @pl.loop(0, _NCHUNK)
    def _(ci):
        off = base + ci * _K
        pltpu.sync_copy(ia_hbm.at[pl.ds(off, _K)], idxa)
        pltpu.sync_copy(ib_hbm.at[pl.ds(off, _K)], idxb)
        ca = pltpu.async_copy(ta_hbm.at[idxa], rowsa, sema)
        cb = pltpu.async_copy(tb_hbm.at[idxb], rowsb, semb)
        ca.wait()
        cb.wait()
        pltpu.sync_copy(rowsa, oa_hbm.at[pl.ds(off, _K)])
        pltpu.sync_copy(rowsb, ob_hbm.at[pl.ds(off, _K)])


_gather_pair = functools.partial(
    pl.kernel,
    mesh=_sc_mesh,
    out_type=(_f32((_E, _OD)), _f32((_E, _OD))),
    scratch_types=[
        pltpu.VMEM((_K,), jnp.int32),
        pltpu.VMEM((_K,), jnp.int32),
        pltpu.VMEM((_K, _OD), jnp.float32),
        pltpu.VMEM((_K, _OD), jnp.float32),
        pltpu.SemaphoreType.DMA,
        pltpu.SemaphoreType.DMA,
    ],
)(_sc_gather_pair_body)


def _sc_scatter_body(with_cnt, *refs):
    if with_cnt:
        (m_hbm, dst_hbm, zeros_hbm, zeros1_hbm,
         out_hbm, cnt_hbm, idxv, rowsv, cntv, acc) = refs
    else:
        (m_hbm, dst_hbm, zeros_hbm,
         out_hbm, idxv, rowsv, acc) = refs
        cntv = None
    cid = lax.axis_index("c")
    sid = lax.axis_index("s")
    wid = sid * _NC + cid
    base = wid * _EPW

    pltpu.sync_copy(zeros_hbm.at[pl.ds(sid * _STRIPE, _STRIPE)],
                    acc.at[pl.ds(sid * _STRIPE, _STRIPE)])
    if with_cnt:
        pltpu.sync_copy(zeros1_hbm, cntv)
    plsc.subcore_barrier()

    ones = jnp.ones((16,), jnp.float32)

    @pl.loop(0, _NCHUNK)
    def _(ci):
        off = base + ci * _K
        pltpu.sync_copy(dst_hbm.at[pl.ds(off, _K)], idxv)
        pltpu.sync_copy(m_hbm.at[pl.ds(off, _K)], rowsv)
        pltpu.sync_copy(rowsv, acc.at[idxv], add=True)
        if with_cnt:
            for j in range(_K // 16):
                idx16 = idxv[pl.ds(j * 16, 16)]
                plsc.addupdate_scatter(cntv, [idx16], ones)

    plsc.subcore_barrier()
    pltpu.sync_copy(acc.at[pl.ds(sid * _STRIPE, _STRIPE)],
                    out_hbm.at[pl.ds(cid * _NP + sid * _STRIPE, _STRIPE)])
    if with_cnt:
        pltpu.sync_copy(cntv, cnt_hbm.at[pl.ds(wid * _NP, _NP)])


def _make_scatter(with_cnt):
    outs = (_f32((_NC * _NP, _OD)), _f32((_NW * _NP,)))
    scratch = [
        pltpu.VMEM((_K,), jnp.int32),
        pltpu.VMEM((_K, _OD), jnp.float32),
    ]
    if with_cnt:
        scratch.append(pltpu.VMEM((_NP,), jnp.float32))
    scratch.append(pltpu.VMEM_SHARED((_NP, _OD), jnp.float32))
    cp = pltpu.CompilerParams()
    if with_cnt and "needs_layout_passes" in pltpu.CompilerParams.__dataclass_fields__:
        cp = dataclasses.replace(cp, needs_layout_passes=False)
    return functools.partial(
        pl.kernel,
        mesh=_sc_mesh,
        out_type=outs if with_cnt else outs[0],
        scratch_types=scratch,
        compiler_params=cp,
    )(functools.partial(_sc_scatter_body, with_cnt))


def _node_pre_body(x_ref, dummy_ref, g0_ref, b0_ref, projw_ref, projb_ref,
                   gatew_ref, gateb_ref, w1ds_ref,
                   pdps_ref, skip_ref, gate_ref):
    x = x_ref[...]
    invalid = x[:, 0:1] == -999.0
    x = jnp.where(invalid, dummy_ref[...], x)
    x = _ln(x, g0_ref[...], b0_ref[...])
    skip = x @ projw_ref[...] + projb_ref[...]
    gate = jax.nn.sigmoid(skip @ gatew_ref[...] + gateb_ref[...])
    skip_ref[...] = skip
    gate_ref[...] = gate
    pdps_ref[...] = x @ w1ds_ref[...]


def _edge_enc_body(ea_ref,
                   lng_ref, lnb_ref, w1_ref, b1_ref, w2_ref, b2_ref,
                   w3_ref, b3_ref, cw1_ref, cb1_ref, cw2_ref, cb2_ref,
                   e1w_ref, e1b_ref,
                   eenc_ref, a1_ref):
    ea = ea_ref[...]
    h = _ln(ea, lng_ref[...], lnb_ref[...])
    h = jnp.maximum(h @ w1_ref[...] + b1_ref[...], 0.0)
    h = jnp.maximum(h @ w2_ref[...] + b2_ref[...], 0.0)
    eenc = h @ w3_ref[...] + b3_ref[...]
    ew = jnp.maximum(ea @ cw1_ref[...] + cb1_ref[...], 0.0)
    ew = jax.nn.sigmoid(ew @ cw2_ref[...] + cb2_ref[...])
    eenc = eenc * ew
    eenc_ref[...] = eenc
    a1_ref[...] = eenc @ e1w_ref[...] + e1b_ref[...]


def _conv1_body(gd_ref, gs_ref, a1_ref,
                w2_ref, b2_ref, w3_ref, b3_ref, m1_ref):
    h = jnp.maximum(gd_ref[:, :_OD // 2] + gs_ref[:, _OD // 2:]
                    + a1_ref[...], 0.0)
    h = jnp.maximum(h @ w2_ref[...] + b2_ref[...], 0.0)
    m1 = h @ w3_ref[...] + b3_ref[...]
    m1_ref[...] = jnp.concatenate(
        [m1, jnp.zeros((_EB, _OD // 2), jnp.float32)], axis=1)


def _conv2_body(eenc_ref, ga_ref, gb_ref,
                we_ref, b1_ref, w2_ref, b2_ref,
                w3_ref, b3_ref, m2_ref):
    h = jnp.maximum(ga_ref[...] + gb_ref[...]
                    + eenc_ref[...] @ we_ref[...] + b1_ref[...], 0.0)
    h = jnp.maximum(h @ w2_ref[...] + b2_ref[...], 0.0)
    m2_ref[...] = h @ w3_ref[...] + b3_ref[...]


def _node_mid_body(s1p_ref, cnt_ref, g_ref, b_ref, w2d_ref, w2s_ref,
                   p2d_ref, p2s_ref):
    s1 = s1p_ref[0:_N, :_OD // 2] + s1p_ref[_NP:_NP + _N, :_OD // 2]
    x1 = s1 / cnt_ref[...]
    x1 = _ln(x1, g_ref[...], b_ref[...])
    x1 = jnp.where(x1 > 0, x1, 0.01 * x1)
    p2d_ref[...] = x1 @ w2d_ref[...]
    p2s_ref[...] = x1 @ w2s_ref[...]


def _node_fin_body(s2p_ref, cnt_ref, efsp_ref, skip_ref, gate_ref,
                   g2_ref, b2_ref, w1_ref, bb1_ref, w2_ref, bb2_ref,
                   w3_ref, bb3_ref, xfc_ref, probs_ref):
    cnt = cnt_ref[...]
    s2 = s2p_ref[0:_N, :] + s2p_ref[_NP:_NP + _N, :]
    x2 = _ln(s2 / cnt, g2_ref[...], b2_ref[...])
    x2 = jnp.maximum(x2, 0.0)
    gate = gate_ref[...]
    xf = gate * skip_ref[...] + (1.0 - gate) * x2
    efm = (efsp_ref[0:_N, :] + efsp_ref[_NP:_NP + _N, :]) / cnt
    xfc = jnp.concatenate([xf, efm], axis=1)
    xfc_ref[...] = xfc
    h = xfc @ w1_ref[...] + bb1_ref[...]
    h = jnp.where(h > 0, h, jnp.exp(jnp.minimum(h, 0.0)) - 1.0)
    h = h @ w2_ref[...] + bb2_ref[...]
    h = jnp.where(h > 0, h, jnp.exp(jnp.minimum(h, 0.0)) - 1.0)
    probs_ref[...] = h @ w3_ref[...] + bb3_ref[...]


@jax.jit
def _run(x_in, edge_index, edge_attr, params):
    p = params
    x = x_in[0]
    ea = edge_attr[0]
    src = edge_index[0, 0]
    dst = edge_index[0, 1]
    r = lambda b: b.reshape(1, -1)

    c1w1 = p['c1_W1']
    w1ds = jnp.concatenate([c1w1[:, :_D].T, c1w1[:, _D:2 * _D].T], axis=1)
    w1e = c1w1[:, 2 * _D:].T
    c2w1 = p['c2_W1']
    w2d = c2w1[:, :_OD // 2].T
    w2s = c2w1[:, _OD // 2:_OD].T
    w2e = c2w1[:, _OD:].T

    zeros128 = jnp.zeros((_NP, _OD), jnp.float32)
    zeros1 = jnp.zeros((_NP,), jnp.float32)

    pdps, skip, gate = pl.pallas_call(
        _node_pre_body,
        out_shape=(_f32((_N, _OD)), _f32((_N, _OD)), _f32((_N, _OD))),
        in_specs=[_full((_N, _D)), _full((1, _D)), _full((1, _D)),
                  _full((1, _D)), _full((_D, _OD)), _full((1, _OD)),
                  _full((_OD, _OD)), _full((1, _OD)),
                  _full((_D, _OD))],
        out_specs=(_full((_N, _OD)), _full((_N, _OD)), _full((_N, _OD))),
        grid=(1,),
    )(x, r(p['dummy']), r(p['bn0_g']), r(p['bn0_b']),
      p['proj_W'].T, r(p['proj_b']), p['gate_W'].T, r(p['gate_b']),
      w1ds)

    nblk = _E // _EB
    eenc, a1 = pl.pallas_call(
        _edge_enc_body,
        out_shape=(_f32((_E, _OD)), _f32((_E, _OD // 2))),
        in_specs=[_eblk(_ED),
                  _full((1, _ED)), _full((1, _ED)),
                  _full((_ED, _OD)), _full((1, _OD)),
                  _full((_OD, 2 * _OD)), _full((1, 2 * _OD)),
                  _full((2 * _OD, _OD)), _full((1, _OD)),
                  _full((_ED, _ED)), _full((1, _ED)),
                  _full((_ED, 1)), _full((1, 1)),
                  _full((_OD, _OD // 2)), _full((1, _OD // 2))],
        out_specs=(_eblk(_OD), _eblk(_OD // 2)),
        grid=(nblk,),
    )(ea, r(p['ee_ln_g']), r(p['ee_ln_b']),
      p['ee_W1'].T, r(p['ee_b1']), p['ee_W2'].T, r(p['ee_b2']),
      p['ee_W3'].T, r(p['ee_b3']),
      p['ec_W1'].T, r(p['ec_b1']), p['ec_W2'].T, r(p['ec_b2']),
      w1e, r(p['c1_b1']))

    gd1, gs1 = _gather_pair(pdps, pdps, dst, src)

    m1 = pl.pallas_call(
        _conv1_body,
        out_shape=_f32((_E, _OD)),
        in_specs=[_eblk(_OD), _eblk(_OD), _eblk(_OD // 2),
                  _full((_OD // 2, _OD // 2)), _full((1, _OD // 2)),
                  _full((_OD // 2, _OD // 2)), _full((1, _OD // 2))],
        out_specs=_eblk(_OD),
        grid=(nblk,),
    )(gd1, gs1, a1,
      p['c1_W2'].T, r(p['c1_b2']), p['c1_W3'].T, r(p['c1_b3']))

    efsp = _make_scatter(False)(eenc, dst, zeros128)

    s1p, cntp = _make_scatter(True)(m1, dst, zeros128, zeros1)

    cnt = jnp.maximum(jnp.sum(cntp.reshape(_NW, _NP)[:, :_N], axis=0), 1.0).reshape(_N, 1)

    p2d, p2s = pl.pallas_call(
        _node_mid_body,
        out_shape=(_f32((_N, _OD)), _f32((_N, _OD))),
        in_specs=[_full((_NC * _NP, _OD)), _full((_N, 1)),
                  _full((1, _OD // 2)), _full((1, _OD // 2)),
                  _full((_OD // 2, _OD)), _full((_OD // 2, _OD))],
        out_specs=(_full((_N, _OD)), _full((_N, _OD))),
        grid=(1,),
    )(s1p, cnt, r(p['bn1_g']), r(p['bn1_b']), w2d, w2s)

    g2d, g2s = _gather_pair(p2d, p2s, dst, src)

    m2 = pl.pallas_call(
        _conv2_body,
        out_shape=_f32((_E, _OD)),
        in_specs=[_eblk(_OD), _eblk(_OD), _eblk(_OD),
                  _full((_OD, _OD)), _full((1, _OD)),
                  _full((_OD, _OD)), _full((1, _OD)),
                  _full((_OD, _OD)), _full((1, _OD))],
        out_specs=_eblk(_OD),
        grid=(nblk,),
    )(eenc, g2d, g2s,
      w2e, r(p['c2_b1']),
      p['c2_W2'].T, r(p['c2_b2']), p['c2_W3'].T, r(p['c2_b3']))

    s2p = _make_scatter(False)(m2, dst, zeros128)

    xfc, probs = pl.pallas_call(
        _node_fin_body,
        out_shape=(_f32((_N, 2 * _OD)), _f32((_N, 1))),
        in_specs=[_full((_NC * _NP, _OD)), _full((_N, 1)),
                  _full((_NC * _NP, _OD)),
                  _full((_N, _OD)), _full((_N, _OD)),
                  _full((1, _OD)), _full((1, _OD)),
                  _full((2 * _OD, _OD)), _full((1, _OD)),
                  _full((_OD, _OD // 2)), _full((1, _OD // 2)),
                  _full((_OD // 2, 1)), _full((1, 1))],
        out_specs=(_full((_N, 2 * _OD)), _full((_N, 1))),
        grid=(1,),
    )(s2p, cnt, efsp, skip, gate,
      r(p['bn2_g']), r(p['bn2_b']),
      p['np_W1'].T, r(p['np_b1']), p['np_W2'].T, r(p['np_b2']),
      p['np_W3'].T, r(p['np_b3']))

    return xfc, probs


def kernel(x_in, edge_index, edge_attr, params):
    return _run(x_in, edge_index, edge_attr, params)


# speedup vs baseline: 3.7873x; 1.3832x over previous
import functools
import jax
import jax.numpy as jnp
from jax import lax
from jax.experimental import pallas as pl
from jax.experimental.pallas import tpu as pltpu
from jax.experimental.pallas import tpu_sc as plsc

_N = 10000
_E = 320000
_D = 128
_OD = 128
_ED = 16
_EB = 3200
_NC, _NS = 2, 16
_NW = _NC * _NS
_EPW = _E // _NW
_K = 80
_NCHUNK = _EPW // _K
_NP = 10240
_STRIPE = _NP // _NS

_sc_mesh = plsc.VectorSubcoreMesh(core_axis_name="c", subcore_axis_name="s")


def _full(shape):
    nd = len(shape)
    return pl.BlockSpec(shape, lambda i: (0,) * nd)


def _eblk(width):
    return pl.BlockSpec((_EB, width), lambda i: (i, 0))


def _f32(shape):
    return jax.ShapeDtypeStruct(shape, jnp.float32)


def _ln(x, g, b, eps=1e-6):
    m = jnp.mean(x, axis=-1, keepdims=True)
    v = jnp.mean((x - m) * (x - m), axis=-1, keepdims=True)
    return (x - m) * jax.lax.rsqrt(v + eps) * g + b


def _sc_gather_pair_body(ta_hbm, tb_hbm, ia_hbm, ib_hbm, oa_hbm, ob_hbm,
                         idxa0, idxb0, idxa1, idxb1,
                         rowsa0, rowsb0, rowsa1, rowsb1,
                         semi0, semi1, semg0, semg1, semw0, semw1):
    wid = lax.axis_index("s") * _NC + lax.axis_index("c")
    base = wid * _EPW
    idxa = (idxa0, idxa1)
    idxb = (idxb0, idxb1)
    rowsa = (rowsa0, rowsa1)
    rowsb = (rowsb0, rowsb1)
    semi = (semi0, semi1)
    semg = (semg0, semg1)
    semw = (semw0, semw1)

    def off(c):
        return base + c * _K

    def issue_i(c, b):
        pltpu.async_copy(ia_hbm.at[pl.ds(off(c), _K)], idxa[b], semi[b])
        pltpu.async_copy(ib_hbm.at[pl.ds(off(c), _K)], idxb[b], semi[b])

    def wait_i(c, b):
        pltpu.make_async_copy(ia_hbm.at[pl.ds(off(c), _K)], idxa[b],
                              semi[b]).wait()
        pltpu.make_async_copy(ib_hbm.at[pl.ds(off(c), _K)], idxb[b],
                              semi[b]).wait()

    def issue_g(b):
        pltpu.async_copy(ta_hbm.at[idxa[b]], rowsa[b], semg[b])
        pltpu.async_copy(tb_hbm.at[idxb[b]], rowsb[b], semg[b])

    def wait_g(b):
        pltpu.make_async_copy(ta_hbm.at[idxa[b]], rowsa[b], semg[b]).wait()
        pltpu.make_async_copy(tb_hbm.at[idxb[b]], rowsb[b], semg[b]).wait()

    def issue_w(c, b):
        pltpu.async_copy(rowsa[b], oa_hbm.at[pl.ds(off(c), _K)], semw[b])
        pltpu.async_copy(rowsb[b], ob_hbm.at[pl.ds(off(c), _K)], semw[b])

    def wait_w(c, b):
        pltpu.make_async_copy(rowsa[b], oa_hbm.at[pl.ds(off(c), _K)],
                              semw[b]).wait()
        pltpu.make_async_copy(rowsb[b], ob_hbm.at[pl.ds(off(c), _K)],
                              semw[b]).wait()

    issue_i(0, 0)
    wait_i(0, 0)
    issue_g(0)
    issue_i(1, 1)
    wait_i(1, 1)
    issue_g(1)
    wait_g(0)
    issue_w(0, 0)
    issue_i(2, 0)

    @pl.loop(0, (_NCHUNK - 3) // 2)
    def _(g):
        for b, d in ((0, 2), (1, 3)):
            c = 2 * g + d
            wait_i(c, b)
            wait_w(c - 2, b)
            issue_g(b)
            wait_g(1 - b)
            issue_w(c - 1, 1 - b)
            issue_i(c + 1, 1 - b)

    c = _NCHUNK - 1
    wait_i(c, 0)
    wait_w(c - 2, 0)
    issue_g(0)
    wait_g(1)
    issue_w(c - 1, 1)
    wait_g(0)
    issue_w(c, 0)
    wait_w(c - 1, 1)
    wait_w(c, 0)


_gather_pair = functools.partial(
    pl.kernel,
    mesh=_sc_mesh,
    out_type=(_f32((_E, _OD)), _f32((_E, _OD))),
    scratch_types=[
        pltpu.VMEM((_K,), jnp.int32),
        pltpu.VMEM((_K,), jnp.int32),
        pltpu.VMEM((_K,), jnp.int32),
        pltpu.VMEM((_K,), jnp.int32),
        pltpu.VMEM((_K, _OD), jnp.float32),
        pltpu.VMEM((_K, _OD), jnp.float32),
        pltpu.VMEM((_K, _OD), jnp.float32),
        pltpu.VMEM((_K, _OD), jnp.float32),
        pltpu.SemaphoreType.DMA,
        pltpu.SemaphoreType.DMA,
        pltpu.SemaphoreType.DMA,
        pltpu.SemaphoreType.DMA,
        pltpu.SemaphoreType.DMA,
        pltpu.SemaphoreType.DMA,
    ],
)(_sc_gather_pair_body)


def _sc_scatter_body(m_hbm, dst_hbm, zeros_hbm, out_hbm,
                     idx0, idx1, rows0, rows1,
                     seml0, seml1, sems0, sems1, acc):
    cid = lax.axis_index("c")
    sid = lax.axis_index("s")
    wid = sid * _NC + cid
    base = wid * _EPW
    idx = (idx0, idx1)
    rows = (rows0, rows1)
    seml = (seml0, seml1)
    sems = (sems0, sems1)

    pltpu.sync_copy(zeros_hbm.at[pl.ds(sid * _STRIPE, _STRIPE)],
                    acc.at[pl.ds(sid * _STRIPE, _STRIPE)])
    plsc.subcore_barrier()

    def off(c):
        return base + c * _K

    def issue_l(c, b):
        pltpu.async_copy(dst_hbm.at[pl.ds(off(c), _K)], idx[b], seml[b])
        pltpu.async_copy(m_hbm.at[pl.ds(off(c), _K)], rows[b], seml[b])

    def wait_l(c, b):
        pltpu.make_async_copy(dst_hbm.at[pl.ds(off(c), _K)], idx[b],
                              seml[b]).wait()
        pltpu.make_async_copy(m_hbm.at[pl.ds(off(c), _K)], rows[b],
                              seml[b]).wait()

    def issue_s(b):
        pltpu.async_copy(rows[b], acc.at[idx[b]], sems[b], add=True)

    def wait_s(b):
        pltpu.make_async_copy(rows[b], acc.at[idx[b]], sems[b]).wait()

    issue_l(0, 0)
    wait_l(0, 0)
    issue_s(0)
    issue_l(1, 1)

    @pl.loop(0, (_NCHUNK - 1) // 2)
    def _(g):
        for b, d in ((1, 1), (0, 2)):
            c = 2 * g + d
            wait_l(c, b)
            issue_s(b)
            wait_s(1 - b)

            @pl.when(c + 1 < _NCHUNK)
            def _():
                issue_l(c + 1, 1 - b)

    wait_s(0)

    plsc.subcore_barrier()
    pltpu.sync_copy(acc.at[pl.ds(sid * _STRIPE, _STRIPE)],
                    out_hbm.at[pl.ds(cid * _NP + sid * _STRIPE, _STRIPE)])


_scatter = functools.partial(
    pl.kernel,
    mesh=_sc_mesh,
    out_type=_f32((_NC * _NP, _OD)),
    scratch_types=[
        pltpu.VMEM((_K,), jnp.int32),
        pltpu.VMEM((_K,), jnp.int32),
        pltpu.VMEM((_K, _OD), jnp.float32),
        pltpu.VMEM((_K, _OD), jnp.float32),
        pltpu.SemaphoreType.DMA,
        pltpu.SemaphoreType.DMA,
        pltpu.SemaphoreType.DMA,
        pltpu.SemaphoreType.DMA,
        pltpu.VMEM_SHARED((_NP, _OD), jnp.float32),
    ],
)(_sc_scatter_body)


def _node_pre_body(x_ref, dummy_ref, g0_ref, b0_ref, projw_ref, projb_ref,
                   gatew_ref, gateb_ref, w1ds_ref,
                   pdps_ref, skip_ref, gate_ref):
    x = x_ref[...]
    invalid = x[:, 0:1] == -999.0
    x = jnp.where(invalid, dummy_ref[...], x)
    x = _ln(x, g0_ref[...], b0_ref[...])
    skip = x @ projw_ref[...] + projb_ref[...]
    gate = jax.nn.sigmoid(skip @ gatew_ref[...] + gateb_ref[...])
    skip_ref[...] = skip
    gate_ref[...] = gate
    pdps_ref[...] = x @ w1ds_ref[...]


def _edge_enc_body(ea_ref,
                   lng_ref, lnb_ref, w1_ref, b1_ref, w2_ref, b2_ref,
                   w3_ref, b3_ref, cw1_ref, cb1_ref, cw2_ref, cb2_ref,
                   e1w_ref, e1b_ref,
                   eenc_ref, a1_ref):
    ea = ea_ref[...]
    h = _ln(ea, lng_ref[...], lnb_ref[...])
    h = jnp.maximum(h @ w1_ref[...] + b1_ref[...], 0.0)
    h = jnp.maximum(h @ w2_ref[...] + b2_ref[...], 0.0)
    eenc = h @ w3_ref[...] + b3_ref[...]
    ew = jnp.maximum(ea @ cw1_ref[...] + cb1_ref[...], 0.0)
    ew = jax.nn.sigmoid(ew @ cw2_ref[...] + cb2_ref[...])
    eenc = eenc * ew
    eenc_ref[...] = eenc
    a1_ref[...] = eenc @ e1w_ref[...] + e1b_ref[...]


def _conv1_body(gd_ref, gs_ref, a1_ref,
                w2_ref, b2_ref, w3_ref, b3_ref, m1_ref):
    h = jnp.maximum(gd_ref[:, :_OD // 2] + gs_ref[:, _OD // 2:]
                    + a1_ref[...], 0.0)
    h = jnp.maximum(h @ w2_ref[...] + b2_ref[...], 0.0)
    m1 = h @ w3_ref[...] + b3_ref[...]
    m1_ref[...] = jnp.concatenate(
        [m1, jnp.zeros((_EB, _OD // 2 - 1), jnp.float32),
         jnp.ones((_EB, 1), jnp.float32)], axis=1)


def _conv2_body(eenc_ref, ga_ref, gb_ref,
                we_ref, b1_ref, w2_ref, b2_ref,
                w3_ref, b3_ref, m2_ref):
    h = jnp.maximum(ga_ref[...] + gb_ref[...]
                    + eenc_ref[...] @ we_ref[...] + b1_ref[...], 0.0)
    h = jnp.maximum(h @ w2_ref[...] + b2_ref[...], 0.0)
    m2_ref[...] = h @ w3_ref[...] + b3_ref[...]


def _node_mid_body(s1p_ref, g_ref, b_ref, w2d_ref, w2s_ref,
                   p2d_ref, p2s_ref, cnt_ref):
    cnt = jnp.maximum(s1p_ref[0:_N, _OD - 1:_OD]
                      + s1p_ref[_NP:_NP + _N, _OD - 1:_OD], 1.0)
    cnt_ref[...] = cnt
    s1 = s1p_ref[0:_N, :_OD // 2] + s1p_ref[_NP:_NP + _N, :_OD // 2]
    x1 = s1 / cnt
    x1 = _ln(x1, g_ref[...], b_ref[...])
    x1 = jnp.where(x1 > 0, x1, 0.01 * x1)
    p2d_ref[...] = x1 @ w2d_ref[...]
    p2s_ref[...] = x1 @ w2s_ref[...]


def _node_fin_body(s2p_ref, cnt_ref, efsp_ref, skip_ref, gate_ref,
                   g2_ref, b2_ref, w1_ref, bb1_ref, w2_ref, bb2_ref,
                   w3_ref, bb3_ref, xfc_ref, probs_ref):
    cnt = cnt_ref[...]
    s2 = s2p_ref[0:_N, :] + s2p_ref[_NP:_NP + _N, :]
    x2 = _ln(s2 / cnt, g2_ref[...], b2_ref[...])
    x2 = jnp.maximum(x2, 0.0)
    gate = gate_ref[...]
    xf = gate * skip_ref[...] + (1.0 - gate) * x2
    efm = (efsp_ref[0:_N, :] + efsp_ref[_NP:_NP + _N, :]) / cnt
    xfc = jnp.concatenate([xf, efm], axis=1)
    xfc_ref[...] = xfc
    h = xfc @ w1_ref[...] + bb1_ref[...]
    h = jnp.where(h > 0, h, jnp.exp(jnp.minimum(h, 0.0)) - 1.0)
    h = h @ w2_ref[...] + bb2_ref[...]
    h = jnp.where(h > 0, h, jnp.exp(jnp.minimum(h, 0.0)) - 1.0)
    probs_ref[...] = h @ w3_ref[...] + bb3_ref[...]


@jax.jit
def _run(x_in, edge_index, edge_attr, params):
    p = params
    x = x_in[0]
    ea = edge_attr[0]
    src = edge_index[0, 0]
    dst = edge_index[0, 1]
    r = lambda b: b.reshape(1, -1)

    c1w1 = p['c1_W1']
    w1ds = jnp.concatenate([c1w1[:, :_D].T, c1w1[:, _D:2 * _D].T], axis=1)
    w1e = c1w1[:, 2 * _D:].T
    c2w1 = p['c2_W1']
    w2d = c2w1[:, :_OD // 2].T
    w2s = c2w1[:, _OD // 2:_OD].T
    w2e = c2w1[:, _OD:].T

    zeros128 = jnp.zeros((_NP, _OD), jnp.float32)

    pdps, skip, gate = pl.pallas_call(
        _node_pre_body,
        out_shape=(_f32((_N, _OD)), _f32((_N, _OD)), _f32((_N, _OD))),
        in_specs=[_full((_N, _D)), _full((1, _D)), _full((1, _D)),
                  _full((1, _D)), _full((_D, _OD)), _full((1, _OD)),
                  _full((_OD, _OD)), _full((1, _OD)),
                  _full((_D, _OD))],
        out_specs=(_full((_N, _OD)), _full((_N, _OD)), _full((_N, _OD))),
        grid=(1,),
    )(x, r(p['dummy']), r(p['bn0_g']), r(p['bn0_b']),
      p['proj_W'].T, r(p['proj_b']), p['gate_W'].T, r(p['gate_b']),
      w1ds)

    nblk = _E // _EB
    eenc, a1 = pl.pallas_call(
        _edge_enc_body,
        out_shape=(_f32((_E, _OD)), _f32((_E, _OD // 2))),
        in_specs=[_eblk(_ED),
                  _full((1, _ED)), _full((1, _ED)),
                  _full((_ED, _OD)), _full((1, _OD)),
                  _full((_OD, 2 * _OD)), _full((1, 2 * _OD)),
                  _full((2 * _OD, _OD)), _full((1, _OD)),
                  _full((_ED, _ED)), _full((1, _ED)),
                  _full((_ED, 1)), _full((1, 1)),
                  _full((_OD, _OD // 2)), _full((1, _OD // 2))],
        out_specs=(_eblk(_OD), _eblk(_OD // 2)),
        grid=(nblk,),
    )(ea, r(p['ee_ln_g']), r(p['ee_ln_b']),
      p['ee_W1'].T, r(p['ee_b1']), p['ee_W2'].T, r(p['ee_b2']),
      p['ee_W3'].T, r(p['ee_b3']),
      p['ec_W1'].T, r(p['ec_b1']), p['ec_W2'].T, r(p['ec_b2']),
      w1e, r(p['c1_b1']))

    gd1, gs1 = _gather_pair(pdps, pdps, dst, src)

    m1 = pl.pallas_call(
        _conv1_body,
        out_shape=_f32((_E, _OD)),
        in_specs=[_eblk(_OD), _eblk(_OD), _eblk(_OD // 2),
                  _full((_OD // 2, _OD // 2)), _full((1, _OD // 2)),
                  _full((_OD // 2, _OD // 2)), _full((1, _OD // 2))],
        out_specs=_eblk(_OD),
        grid=(nblk,),
    )(gd1, gs1, a1,
      p['c1_W2'].T, r(p['c1_b2']), p['c1_W3'].T, r(p['c1_b3']))

    efsp = _scatter(eenc, dst, zeros128)

    s1p = _scatter(m1, dst, zeros128)

    p2d, p2s, cnt = pl.pallas_call(
        _node_mid_body,
        out_shape=(_f32((_N, _OD)), _f32((_N, _OD)), _f32((_N, 1))),
        in_specs=[_full((_NC * _NP, _OD)),
                  _full((1, _OD // 2)), _full((1, _OD // 2)),
                  _full((_OD // 2, _OD)), _full((_OD // 2, _OD))],
        out_specs=(_full((_N, _OD)), _full((_N, _OD)), _full((_N, 1))),
        grid=(1,),
    )(s1p, r(p['bn1_g']), r(p['bn1_b']), w2d, w2s)

    g2d, g2s = _gather_pair(p2d, p2s, dst, src)

    m2 = pl.pallas_call(
        _conv2_body,
        out_shape=_f32((_E, _OD)),
        in_specs=[_eblk(_OD), _eblk(_OD), _eblk(_OD),
                  _full((_OD, _OD)), _full((1, _OD)),
                  _full((_OD, _OD)), _full((1, _OD)),
                  _full((_OD, _OD)), _full((1, _OD))],
        out_specs=_eblk(_OD),
        grid=(nblk,),
    )(eenc, g2d, g2s,
      w2e, r(p['c2_b1']),
      p['c2_W2'].T, r(p['c2_b2']), p['c2_W3'].T, r(p['c2_b3']))

    s2p = _scatter(m2, dst, zeros128)

    xfc, probs = pl.pallas_call(
        _node_fin_body,
        out_shape=(_f32((_N, 2 * _OD)), _f32((_N, 1))),
        in_specs=[_full((_NC * _NP, _OD)), _full((_N, 1)),
                  _full((_NC * _NP, _OD)),
                  _full((_N, _OD)), _full((_N, _OD)),
                  _full((1, _OD)), _full((1, _OD)),
                  _full((2 * _OD, _OD)), _full((1, _OD)),
                  _full((_OD, _OD // 2)), _full((1, _OD // 2)),
                  _full((_OD // 2, 1)), _full((1, 1))],
        out_specs=(_full((_N, 2 * _OD)), _full((_N, 1))),
        grid=(1,),
    )(s2p, cnt, efsp, skip, gate,
      r(p['bn2_g']), r(p['bn2_b']),
      p['np_W1'].T, r(p['np_b1']), p['np_W2'].T, r(p['np_b2']),
      p['np_W3'].T, r(p['np_b3']))

    return xfc, probs


def kernel(x_in, edge_index, edge_attr, params):
    return _run(x_in, edge_index, edge_attr, params)


# speedup vs baseline: 4.3425x; 1.1466x over previous
import functools
import jax
import jax.numpy as jnp
from jax import lax
from jax.experimental import pallas as pl
from jax.experimental.pallas import tpu as pltpu
from jax.experimental.pallas import tpu_sc as plsc

_N = 10000
_E = 320000
_D = 128
_OD = 128
_ED = 16
_EB = 3200
_NC, _NS = 2, 16
_NW = _NC * _NS
_EPW = _E // _NW
_K = 80
_NCHUNK = _EPW // _K
_NP = 10240
_STRIPE = _NP // _NS

_sc_mesh = plsc.VectorSubcoreMesh(core_axis_name="c", subcore_axis_name="s")


def _full(shape):
    nd = len(shape)
    return pl.BlockSpec(shape, lambda i: (0,) * nd)


def _eblk(width):
    return pl.BlockSpec((_EB, width), lambda i: (i, 0))


def _f32(shape):
    return jax.ShapeDtypeStruct(shape, jnp.float32)


def _ln(x, g, b, eps=1e-6):
    m = jnp.mean(x, axis=-1, keepdims=True)
    v = jnp.mean((x - m) * (x - m), axis=-1, keepdims=True)
    return (x - m) * jax.lax.rsqrt(v + eps) * g + b


def _sc_gather_pair_body(ta_hbm, tb_hbm, ia_hbm, ib_hbm, o_hbm,
                         idxa0, idxb0, idxa1, idxb1,
                         rowsa0, rowsb0, rowsa1, rowsb1,
                         semi0, semi1, semg0, semg1, semw0, semw1):
    wid = lax.axis_index("s") * _NC + lax.axis_index("c")
    base = wid * _EPW
    idxa = (idxa0, idxa1)
    idxb = (idxb0, idxb1)
    rowsa = (rowsa0, rowsa1)
    rowsb = (rowsb0, rowsb1)
    semi = (semi0, semi1)
    semg = (semg0, semg1)
    semw = (semw0, semw1)

    def off(c):
        return base + c * _K

    def issue_i(c, b):
        pltpu.async_copy(ia_hbm.at[pl.ds(off(c), _K)], idxa[b], semi[b])
        pltpu.async_copy(ib_hbm.at[pl.ds(off(c), _K)], idxb[b], semi[b])

    def wait_i(c, b):
        pltpu.make_async_copy(ia_hbm.at[pl.ds(off(c), _K)], idxa[b],
                              semi[b]).wait()
        pltpu.make_async_copy(ib_hbm.at[pl.ds(off(c), _K)], idxb[b],
                              semi[b]).wait()

    def issue_g(b):
        pltpu.async_copy(ta_hbm.at[idxa[b]], rowsa[b], semg[b])
        pltpu.async_copy(tb_hbm.at[idxb[b]], rowsb[b], semg[b])

    def wait_g(b):
        pltpu.make_async_copy(ta_hbm.at[idxa[b]], rowsa[b], semg[b]).wait()
        pltpu.make_async_copy(tb_hbm.at[idxb[b]], rowsb[b], semg[b]).wait()

    def issue_w(c, b):
        pltpu.async_copy(rowsa[b], o_hbm.at[pl.ds(off(c), _K)], semw[b])

    def wait_w(c, b):
        pltpu.make_async_copy(rowsa[b], o_hbm.at[pl.ds(off(c), _K)],
                              semw[b]).wait()

    def splice(b):
        ra = rowsa[b]
        rb = rowsb[b]

        @pl.loop(0, _K)
        def _(j):
            for k in range(_OD // 32):
                sl = pl.ds(_OD // 2 + 16 * k, 16)
                ra[j, sl] = rb[j, sl]

    issue_i(0, 0)
    wait_i(0, 0)
    issue_g(0)
    issue_i(1, 1)
    wait_i(1, 1)
    issue_g(1)
    wait_g(0)
    splice(0)
    issue_w(0, 0)
    issue_i(2, 0)

    @pl.loop(0, (_NCHUNK - 3) // 2)
    def _(g):
        for b, d in ((0, 2), (1, 3)):
            c = 2 * g + d
            wait_i(c, b)
            wait_w(c - 2, b)
            issue_g(b)
            wait_g(1 - b)
            splice(1 - b)
            issue_w(c - 1, 1 - b)
            issue_i(c + 1, 1 - b)

    c = _NCHUNK - 1
    wait_i(c, 0)
    wait_w(c - 2, 0)
    issue_g(0)
    wait_g(1)
    splice(1)
    issue_w(c - 1, 1)
    wait_g(0)
    splice(0)
    issue_w(c, 0)
    wait_w(c - 1, 1)
    wait_w(c, 0)


_gather_pair = functools.partial(
    pl.kernel,
    mesh=_sc_mesh,
    out_type=_f32((_E, _OD)),
    scratch_types=[
        pltpu.VMEM((_K,), jnp.int32),
        pltpu.VMEM((_K,), jnp.int32),
        pltpu.VMEM((_K,), jnp.int32),
        pltpu.VMEM((_K,), jnp.int32),
        pltpu.VMEM((_K, _OD), jnp.float32),
        pltpu.VMEM((_K, _OD), jnp.float32),
        pltpu.VMEM((_K, _OD), jnp.float32),
        pltpu.VMEM((_K, _OD), jnp.float32),
        pltpu.SemaphoreType.DMA,
        pltpu.SemaphoreType.DMA,
        pltpu.SemaphoreType.DMA,
        pltpu.SemaphoreType.DMA,
        pltpu.SemaphoreType.DMA,
        pltpu.SemaphoreType.DMA,
    ],
)(_sc_gather_pair_body)


def _sc_scatter_body(m_hbm, dst_hbm, zeros_hbm, out_hbm,
                     idx0, idx1, rows0, rows1,
                     seml0, seml1, sems0, sems1, acc):
    cid = lax.axis_index("c")
    sid = lax.axis_index("s")
    wid = sid * _NC + cid
    base = wid * _EPW
    idx = (idx0, idx1)
    rows = (rows0, rows1)
    seml = (seml0, seml1)
    sems = (sems0, sems1)

    pltpu.sync_copy(zeros_hbm.at[pl.ds(sid * _STRIPE, _STRIPE)],
                    acc.at[pl.ds(sid * _STRIPE, _STRIPE)])
    plsc.subcore_barrier()

    def off(c):
        return base + c * _K

    def issue_l(c, b):
        pltpu.async_copy(dst_hbm.at[pl.ds(off(c), _K)], idx[b], seml[b])
        pltpu.async_copy(m_hbm.at[pl.ds(off(c), _K)], rows[b], seml[b])

    def wait_l(c, b):
        pltpu.make_async_copy(dst_hbm.at[pl.ds(off(c), _K)], idx[b],
                              seml[b]).wait()
        pltpu.make_async_copy(m_hbm.at[pl.ds(off(c), _K)], rows[b],
                              seml[b]).wait()

    def issue_s(b):
        pltpu.async_copy(rows[b], acc.at[idx[b]], sems[b], add=True)

    def wait_s(b):
        pltpu.make_async_copy(rows[b], acc.at[idx[b]], sems[b]).wait()

    issue_l(0, 0)
    wait_l(0, 0)
    issue_s(0)
    issue_l(1, 1)

    @pl.loop(0, (_NCHUNK - 1) // 2)
    def _(g):
        for b, d in ((1, 1), (0, 2)):
            c = 2 * g + d
            wait_l(c, b)
            issue_s(b)
            wait_s(1 - b)

            @pl.when(c + 1 < _NCHUNK)
            def _():
                issue_l(c + 1, 1 - b)

    wait_s(0)

    plsc.subcore_barrier()
    pltpu.sync_copy(acc.at[pl.ds(sid * _STRIPE, _STRIPE)],
                    out_hbm.at[pl.ds(cid * _NP + sid * _STRIPE, _STRIPE)])


_scatter = functools.partial(
    pl.kernel,
    mesh=_sc_mesh,
    out_type=_f32((_NC * _NP, _OD)),
    scratch_types=[
        pltpu.VMEM((_K,), jnp.int32),
        pltpu.VMEM((_K,), jnp.int32),
        pltpu.VMEM((_K, _OD), jnp.float32),
        pltpu.VMEM((_K, _OD), jnp.float32),
        pltpu.SemaphoreType.DMA,
        pltpu.SemaphoreType.DMA,
        pltpu.SemaphoreType.DMA,
        pltpu.SemaphoreType.DMA,
        pltpu.VMEM_SHARED((_NP, _OD), jnp.float32),
    ],
)(_sc_scatter_body)


def _node_pre_body(x_ref, dummy_ref, g0_ref, b0_ref, projw_ref, projb_ref,
                   gatew_ref, gateb_ref, w1ds_ref,
                   pdps_ref, skip_ref, gate_ref):
    x = x_ref[...]
    invalid = x[:, 0:1] == -999.0
    x = jnp.where(invalid, dummy_ref[...], x)
    x = _ln(x, g0_ref[...], b0_ref[...])
    skip = x @ projw_ref[...] + projb_ref[...]
    gate = jax.nn.sigmoid(skip @ gatew_ref[...] + gateb_ref[...])
    skip_ref[...] = skip
    gate_ref[...] = gate
    pdps_ref[...] = x @ w1ds_ref[...]


def _edge_enc_body(ea_ref,
                   lng_ref, lnb_ref, w1_ref, b1_ref, w2_ref, b2_ref,
                   w3_ref, b3_ref, cw1_ref, cb1_ref, cw2_ref, cb2_ref,
                   eenc_ref):
    ea = ea_ref[...]
    h = _ln(ea, lng_ref[...], lnb_ref[...])
    h = jnp.maximum(h @ w1_ref[...] + b1_ref[...], 0.0)
    h = jnp.maximum(h @ w2_ref[...] + b2_ref[...], 0.0)
    eenc = h @ w3_ref[...] + b3_ref[...]
    ew = jnp.maximum(ea @ cw1_ref[...] + cb1_ref[...], 0.0)
    ew = jax.nn.sigmoid(ew @ cw2_ref[...] + cb2_ref[...])
    eenc_ref[...] = eenc * ew


def _conv1_body(g1_ref, eenc_ref, e1w_ref, e1b_ref,
                w2_ref, b2_ref, w3_ref, b3_ref, m1_ref):
    a1 = eenc_ref[...] @ e1w_ref[...] + e1b_ref[...]
    h = jnp.maximum(g1_ref[:, :_OD // 2] + g1_ref[:, _OD // 2:] + a1, 0.0)
    h = jnp.maximum(h @ w2_ref[...] + b2_ref[...], 0.0)
    m1 = h @ w3_ref[...] + b3_ref[...]
    m1_ref[...] = jnp.concatenate(
        [m1, jnp.zeros((_EB, _OD // 2 - 1), jnp.float32),
         jnp.ones((_EB, 1), jnp.float32)], axis=1)


def _conv2_body(eenc_ref, g2_ref, wds_ref,
                we_ref, b1_ref, w2_ref, b2_ref,
                w3_ref, b3_ref, m2_ref):
    h = jnp.maximum(g2_ref[...] @ wds_ref[...]
                    + eenc_ref[...] @ we_ref[...] + b1_ref[...], 0.0)
    h = jnp.maximum(h @ w2_ref[...] + b2_ref[...], 0.0)
    m2_ref[...] = h @ w3_ref[...] + b3_ref[...]


def _node_mid_body(s1p_ref, g_ref, b_ref,
                   t2_ref, cnt_ref):
    cnt = jnp.maximum(s1p_ref[0:_N, _OD - 1:_OD]
                      + s1p_ref[_NP:_NP + _N, _OD - 1:_OD], 1.0)
    cnt_ref[...] = cnt
    s1 = s1p_ref[0:_N, :_OD // 2] + s1p_ref[_NP:_NP + _N, :_OD // 2]
    x1 = s1 / cnt
    x1 = _ln(x1, g_ref[...], b_ref[...])
    x1 = jnp.where(x1 > 0, x1, 0.01 * x1)
    t2_ref[...] = jnp.concatenate([x1, x1], axis=1)


def _node_fin_body(s2p_ref, cnt_ref, efsp_ref, skip_ref, gate_ref,
                   g2_ref, b2_ref, w1_ref, bb1_ref, w2_ref, bb2_ref,
                   w3_ref, bb3_ref, xfc_ref, probs_ref):
    cnt = cnt_ref[...]
    s2 = s2p_ref[0:_N, :] + s2p_ref[_NP:_NP + _N, :]
    x2 = _ln(s2 / cnt, g2_ref[...], b2_ref[...])
    x2 = jnp.maximum(x2, 0.0)
    gate = gate_ref[...]
    xf = gate * skip_ref[...] + (1.0 - gate) * x2
    efm = (efsp_ref[0:_N, :] + efsp_ref[_NP:_NP + _N, :]) / cnt
    xfc = jnp.concatenate([xf, efm], axis=1)
    xfc_ref[...] = xfc
    h = xfc @ w1_ref[...] + bb1_ref[...]
    h = jnp.where(h > 0, h, jnp.exp(jnp.minimum(h, 0.0)) - 1.0)
    h = h @ w2_ref[...] + bb2_ref[...]
    h = jnp.where(h > 0, h, jnp.exp(jnp.minimum(h, 0.0)) - 1.0)
    probs_ref[...] = h @ w3_ref[...] + bb3_ref[...]


@jax.jit
def _run(x_in, edge_index, edge_attr, params):
    p = params
    x = x_in[0]
    ea = edge_attr[0]
    src = edge_index[0, 0]
    dst = edge_index[0, 1]
    r = lambda b: b.reshape(1, -1)

    c1w1 = p['c1_W1']
    w1ds = jnp.concatenate([c1w1[:, :_D].T, c1w1[:, _D:2 * _D].T], axis=1)
    w1e = c1w1[:, 2 * _D:].T
    c2w1 = p['c2_W1']
    w2ds = jnp.concatenate([c2w1[:, :_OD // 2].T,
                            c2w1[:, _OD // 2:_OD].T], axis=0)
    w2e = c2w1[:, _OD:].T

    zeros128 = jnp.zeros((_NP, _OD), jnp.float32)

    pdps, skip, gate = pl.pallas_call(
        _node_pre_body,
        out_shape=(_f32((_N, _OD)), _f32((_N, _OD)), _f32((_N, _OD))),
        in_specs=[_full((_N, _D)), _full((1, _D)), _full((1, _D)),
                  _full((1, _D)), _full((_D, _OD)), _full((1, _OD)),
                  _full((_OD, _OD)), _full((1, _OD)),
                  _full((_D, _OD))],
        out_specs=(_full((_N, _OD)), _full((_N, _OD)), _full((_N, _OD))),
        grid=(1,),
    )(x, r(p['dummy']), r(p['bn0_g']), r(p['bn0_b']),
      p['proj_W'].T, r(p['proj_b']), p['gate_W'].T, r(p['gate_b']),
      w1ds)

    nblk = _E // _EB
    eenc = pl.pallas_call(
        _edge_enc_body,
        out_shape=_f32((_E, _OD)),
        in_specs=[_eblk(_ED),
                  _full((1, _ED)), _full((1, _ED)),
                  _full((_ED, _OD)), _full((1, _OD)),
                  _full((_OD, 2 * _OD)), _full((1, 2 * _OD)),
                  _full((2 * _OD, _OD)), _full((1, _OD)),
                  _full((_ED, _ED)), _full((1, _ED)),
                  _full((_ED, 1)), _full((1, 1))],
        out_specs=_eblk(_OD),
        grid=(nblk,),
    )(ea, r(p['ee_ln_g']), r(p['ee_ln_b']),
      p['ee_W1'].T, r(p['ee_b1']), p['ee_W2'].T, r(p['ee_b2']),
      p['ee_W3'].T, r(p['ee_b3']),
      p['ec_W1'].T, r(p['ec_b1']), p['ec_W2'].T, r(p['ec_b2']))

    g1 = _gather_pair(pdps, pdps, dst, src)

    m1 = pl.pallas_call(
        _conv1_body,
        out_shape=_f32((_E, _OD)),
        in_specs=[_eblk(_OD), _eblk(_OD),
                  _full((_OD, _OD // 2)), _full((1, _OD // 2)),
                  _full((_OD // 2, _OD // 2)), _full((1, _OD // 2)),
                  _full((_OD // 2, _OD // 2)), _full((1, _OD // 2))],
        out_specs=_eblk(_OD),
        grid=(nblk,),
    )(g1, eenc, w1e, r(p['c1_b1']),
      p['c1_W2'].T, r(p['c1_b2']), p['c1_W3'].T, r(p['c1_b3']))

    s1p = _scatter(m1, dst, zeros128)

    t2, cnt = pl.pallas_call(
        _node_mid_body,
        out_shape=(_f32((_N, _OD)), _f32((_N, 1))),
        in_specs=[_full((_NC * _NP, _OD)),
                  _full((1, _OD // 2)), _full((1, _OD // 2))],
        out_specs=(_full((_N, _OD)), _full((_N, 1))),
        grid=(1,),
    )(s1p, r(p['bn1_g']), r(p['bn1_b']))

    g2 = _gather_pair(t2, t2, dst, src)

    efsp = _scatter(eenc, dst, zeros128)

    m2 = pl.pallas_call(
        _conv2_body,
        out_shape=_f32((_E, _OD)),
        in_specs=[_eblk(_OD), _eblk(_OD),
                  _full((_OD, _OD)),
                  _full((_OD, _OD)), _full((1, _OD)),
                  _full((_OD, _OD)), _full((1, _OD)),
                  _full((_OD, _OD)), _full((1, _OD))],
        out_specs=_eblk(_OD),
        grid=(nblk,),
    )(eenc, g2, w2ds,
      w2e, r(p['c2_b1']),
      p['c2_W2'].T, r(p['c2_b2']), p['c2_W3'].T, r(p['c2_b3']))

    s2p = _scatter(m2, dst, zeros128)

    xfc, probs = pl.pallas_call(
        _node_fin_body,
        out_shape=(_f32((_N, 2 * _OD)), _f32((_N, 1))),
        in_specs=[_full((_NC * _NP, _OD)), _full((_N, 1)),
                  _full((_NC * _NP, _OD)),
                  _full((_N, _OD)), _full((_N, _OD)),
                  _full((1, _OD)), _full((1, _OD)),
                  _full((2 * _OD, _OD)), _full((1, _OD)),
                  _full((_OD, _OD // 2)), _full((1, _OD // 2)),
                  _full((_OD // 2, 1)), _full((1, 1))],
        out_specs=(_full((_N, 2 * _OD)), _full((_N, 1))),
        grid=(1,),
    )(s2p, cnt, efsp, skip, gate,
      r(p['bn2_g']), r(p['bn2_b']),
      p['np_W1'].T, r(p['np_b1']), p['np_W2'].T, r(p['np_b2']),
      p['np_W3'].T, r(p['np_b3']))

    return xfc, probs


def kernel(x_in, edge_index, edge_attr, params):
    return _run(x_in, edge_index, edge_attr, params)


# speedup vs baseline: 4.3521x; 1.0022x over previous
import functools
import jax
import jax.numpy as jnp
from jax import lax
from jax.experimental import pallas as pl
from jax.experimental.pallas import tpu as pltpu
from jax.experimental.pallas import tpu_sc as plsc

_N = 10000
_E = 320000
_D = 128
_OD = 128
_ED = 16
_EB = 3200
_NC, _NS = 2, 16
_NW = _NC * _NS
_EPW = _E // _NW
_K = 80
_NCHUNK = _EPW // _K
_NP = 10240
_STRIPE = _NP // _NS

_sc_mesh = plsc.VectorSubcoreMesh(core_axis_name="c", subcore_axis_name="s")


def _full(shape):
    nd = len(shape)
    return pl.BlockSpec(shape, lambda i: (0,) * nd)


def _eblk(width):
    return pl.BlockSpec((_EB, width), lambda i: (i, 0))


def _f32(shape):
    return jax.ShapeDtypeStruct(shape, jnp.float32)


def _ln(x, g, b, eps=1e-6):
    m = jnp.mean(x, axis=-1, keepdims=True)
    v = jnp.mean((x - m) * (x - m), axis=-1, keepdims=True)
    return (x - m) * jax.lax.rsqrt(v + eps) * g + b


def _sc_gather_pair_body(ta_hbm, tb_hbm, ia_hbm, ib_hbm, o_hbm,
                         idxa0, idxb0, idxa1, idxb1,
                         rowsa0, rowsb0, rowsa1, rowsb1,
                         semi0, semi1, semg0, semg1, semw0, semw1):
    wid = lax.axis_index("s") * _NC + lax.axis_index("c")
    base = wid * _EPW
    idxa = (idxa0, idxa1)
    idxb = (idxb0, idxb1)
    rowsa = (rowsa0, rowsa1)
    rowsb = (rowsb0, rowsb1)
    semi = (semi0, semi1)
    semg = (semg0, semg1)
    semw = (semw0, semw1)

    def off(c):
        return base + c * _K

    def issue_i(c, b):
        pltpu.async_copy(ia_hbm.at[pl.ds(off(c), _K)], idxa[b], semi[b])
        pltpu.async_copy(ib_hbm.at[pl.ds(off(c), _K)], idxb[b], semi[b])

    def wait_i(c, b):
        pltpu.make_async_copy(ia_hbm.at[pl.ds(off(c), _K)], idxa[b],
                              semi[b]).wait()
        pltpu.make_async_copy(ib_hbm.at[pl.ds(off(c), _K)], idxb[b],
                              semi[b]).wait()

    def issue_g(b):
        pltpu.async_copy(ta_hbm.at[idxa[b]], rowsa[b], semg[b])
        pltpu.async_copy(tb_hbm.at[idxb[b]], rowsb[b], semg[b])

    def wait_g(b):
        pltpu.make_async_copy(ta_hbm.at[idxa[b]], rowsa[b], semg[b]).wait()
        pltpu.make_async_copy(tb_hbm.at[idxb[b]], rowsb[b], semg[b]).wait()

    def issue_w(c, b):
        pltpu.async_copy(rowsa[b], o_hbm.at[pl.ds(off(c), _K)], semw[b])

    def wait_w(c, b):
        pltpu.make_async_copy(rowsa[b], o_hbm.at[pl.ds(off(c), _K)],
                              semw[b]).wait()

    def splice(b):
        ra = rowsa[b]
        rb = rowsb[b]

        @pl.loop(0, _K, step=4)
        def _(j):
            for jj in range(4):
                for k in range(_OD // 32):
                    sl = pl.ds(_OD // 2 + 16 * k, 16)
                    ra[j + jj, sl] = rb[j + jj, sl]

    issue_i(0, 0)
    wait_i(0, 0)
    issue_g(0)
    issue_i(1, 1)
    wait_i(1, 1)
    issue_g(1)
    wait_g(0)
    splice(0)
    issue_w(0, 0)
    issue_i(2, 0)

    @pl.loop(0, (_NCHUNK - 3) // 2)
    def _(g):
        for b, d in ((0, 2), (1, 3)):
            c = 2 * g + d
            wait_i(c, b)
            wait_w(c - 2, b)
            issue_g(b)
            wait_g(1 - b)
            issue_i(c + 1, 1 - b)
            splice(1 - b)
            issue_w(c - 1, 1 - b)

    c = _NCHUNK - 1
    wait_i(c, 0)
    wait_w(c - 2, 0)
    issue_g(0)
    wait_g(1)
    splice(1)
    issue_w(c - 1, 1)
    wait_g(0)
    splice(0)
    issue_w(c, 0)
    wait_w(c - 1, 1)
    wait_w(c, 0)


_gather_pair = functools.partial(
    pl.kernel,
    mesh=_sc_mesh,
    out_type=_f32((_E, _OD)),
    scratch_types=[
        pltpu.VMEM((_K,), jnp.int32),
        pltpu.VMEM((_K,), jnp.int32),
        pltpu.VMEM((_K,), jnp.int32),
        pltpu.VMEM((_K,), jnp.int32),
        pltpu.VMEM((_K, _OD), jnp.float32),
        pltpu.VMEM((_K, _OD), jnp.float32),
        pltpu.VMEM((_K, _OD), jnp.float32),
        pltpu.VMEM((_K, _OD), jnp.float32),
        pltpu.SemaphoreType.DMA,
        pltpu.SemaphoreType.DMA,
        pltpu.SemaphoreType.DMA,
        pltpu.SemaphoreType.DMA,
        pltpu.SemaphoreType.DMA,
        pltpu.SemaphoreType.DMA,
    ],
)(_sc_gather_pair_body)


def _sc_scatter_body(m_hbm, dst_hbm, zeros_hbm, out_hbm,
                     idx0, idx1, rows0, rows1,
                     seml0, seml1, sems0, sems1, acc):
    cid = lax.axis_index("c")
    sid = lax.axis_index("s")
    wid = sid * _NC + cid
    base = wid * _EPW
    idx = (idx0, idx1)
    rows = (rows0, rows1)
    seml = (seml0, seml1)
    sems = (sems0, sems1)

    pltpu.sync_copy(zeros_hbm.at[pl.ds(sid * _STRIPE, _STRIPE)],
                    acc.at[pl.ds(sid * _STRIPE, _STRIPE)])
    plsc.subcore_barrier()

    def off(c):
        return base + c * _K

    def issue_l(c, b):
        pltpu.async_copy(dst_hbm.at[pl.ds(off(c), _K)], idx[b], seml[b])
        pltpu.async_copy(m_hbm.at[pl.ds(off(c), _K)], rows[b], seml[b])

    def wait_l(c, b):
        pltpu.make_async_copy(dst_hbm.at[pl.ds(off(c), _K)], idx[b],
                              seml[b]).wait()
        pltpu.make_async_copy(m_hbm.at[pl.ds(off(c), _K)], rows[b],
                              seml[b]).wait()

    def issue_s(b):
        pltpu.async_copy(rows[b], acc.at[idx[b]], sems[b], add=True)

    def wait_s(b):
        pltpu.make_async_copy(rows[b], acc.at[idx[b]], sems[b]).wait()

    issue_l(0, 0)
    wait_l(0, 0)
    issue_s(0)
    issue_l(1, 1)

    @pl.loop(0, (_NCHUNK - 1) // 2)
    def _(g):
        for b, d in ((1, 1), (0, 2)):
            c = 2 * g + d
            wait_l(c, b)
            issue_s(b)
            wait_s(1 - b)

            @pl.when(c + 1 < _NCHUNK)
            def _():
                issue_l(c + 1, 1 - b)

    wait_s(0)

    plsc.subcore_barrier()
    pltpu.sync_copy(acc.at[pl.ds(sid * _STRIPE, _STRIPE)],
                    out_hbm.at[pl.ds(cid * _NP + sid * _STRIPE, _STRIPE)])


_scatter = functools.partial(
    pl.kernel,
    mesh=_sc_mesh,
    out_type=_f32((_NC * _NP, _OD)),
    scratch_types=[
        pltpu.VMEM((_K,), jnp.int32),
        pltpu.VMEM((_K,), jnp.int32),
        pltpu.VMEM((_K, _OD), jnp.float32),
        pltpu.VMEM((_K, _OD), jnp.float32),
        pltpu.SemaphoreType.DMA,
        pltpu.SemaphoreType.DMA,
        pltpu.SemaphoreType.DMA,
        pltpu.SemaphoreType.DMA,
        pltpu.VMEM_SHARED((_NP, _OD), jnp.float32),
    ],
)(_sc_scatter_body)


def _node_pre_body(x_ref, dummy_ref, g0_ref, b0_ref, projw_ref, projb_ref,
                   gatew_ref, gateb_ref, w1ds_ref,
                   pdps_ref, skip_ref, gate_ref):
    x = x_ref[...]
    invalid = x[:, 0:1] == -999.0
    x = jnp.where(invalid, dummy_ref[...], x)
    x = _ln(x, g0_ref[...], b0_ref[...])
    skip = x @ projw_ref[...] + projb_ref[...]
    gate = jax.nn.sigmoid(skip @ gatew_ref[...] + gateb_ref[...])
    skip_ref[...] = skip
    gate_ref[...] = gate
    pdps_ref[...] = x @ w1ds_ref[...]


def _edge_enc_body(ea_ref,
                   lng_ref, lnb_ref, w1_ref, b1_ref, w2_ref, b2_ref,
                   w3_ref, b3_ref, cw1_ref, cb1_ref, cw2_ref, cb2_ref,
                   eenc_ref):
    ea = ea_ref[...]
    h = _ln(ea, lng_ref[...], lnb_ref[...])
    h = jnp.maximum(h @ w1_ref[...] + b1_ref[...], 0.0)
    h = jnp.maximum(h @ w2_ref[...] + b2_ref[...], 0.0)
    eenc = h @ w3_ref[...] + b3_ref[...]
    ew = jnp.maximum(ea @ cw1_ref[...] + cb1_ref[...], 0.0)
    ew = jax.nn.sigmoid(ew @ cw2_ref[...] + cb2_ref[...])
    eenc_ref[...] = eenc * ew


def _conv1_body(g1_ref, eenc_ref, e1w_ref, e1b_ref,
                w2_ref, b2_ref, w3_ref, b3_ref, m1_ref):
    a1 = eenc_ref[...] @ e1w_ref[...] + e1b_ref[...]
    h = jnp.maximum(g1_ref[:, :_OD // 2] + g1_ref[:, _OD // 2:] + a1, 0.0)
    h = jnp.maximum(h @ w2_ref[...] + b2_ref[...], 0.0)
    m1 = h @ w3_ref[...] + b3_ref[...]
    m1_ref[...] = jnp.concatenate(
        [m1, jnp.zeros((_EB, _OD // 2 - 1), jnp.float32),
         jnp.ones((_EB, 1), jnp.float32)], axis=1)


def _conv2_body(eenc_ref, g2_ref, wds_ref,
                we_ref, b1_ref, w2_ref, b2_ref,
                w3_ref, b3_ref, m2_ref):
    h = jnp.maximum(g2_ref[...] @ wds_ref[...]
                    + eenc_ref[...] @ we_ref[...] + b1_ref[...], 0.0)
    h = jnp.maximum(h @ w2_ref[...] + b2_ref[...], 0.0)
    m2_ref[...] = h @ w3_ref[...] + b3_ref[...]


def _node_mid_body(s1p_ref, g_ref, b_ref,
                   t2_ref, cnt_ref):
    cnt = jnp.maximum(s1p_ref[0:_N, _OD - 1:_OD]
                      + s1p_ref[_NP:_NP + _N, _OD - 1:_OD], 1.0)
    cnt_ref[...] = cnt
    s1 = s1p_ref[0:_N, :_OD // 2] + s1p_ref[_NP:_NP + _N, :_OD // 2]
    x1 = s1 / cnt
    x1 = _ln(x1, g_ref[...], b_ref[...])
    x1 = jnp.where(x1 > 0, x1, 0.01 * x1)
    t2_ref[...] = jnp.concatenate([x1, x1], axis=1)


def _node_fin_body(s2p_ref, cnt_ref, efsp_ref, skip_ref, gate_ref,
                   g2_ref, b2_ref, w1_ref, bb1_ref, w2_ref, bb2_ref,
                   w3_ref, bb3_ref, xfc_ref, probs_ref):
    cnt = cnt_ref[...]
    s2 = s2p_ref[0:_N, :] + s2p_ref[_NP:_NP + _N, :]
    x2 = _ln(s2 / cnt, g2_ref[...], b2_ref[...])
    x2 = jnp.maximum(x2, 0.0)
    gate = gate_ref[...]
    xf = gate * skip_ref[...] + (1.0 - gate) * x2
    efm = (efsp_ref[0:_N, :] + efsp_ref[_NP:_NP + _N, :]) / cnt
    xfc = jnp.concatenate([xf, efm], axis=1)
    xfc_ref[...] = xfc
    h = xfc @ w1_ref[...] + bb1_ref[...]
    h = jnp.where(h > 0, h, jnp.exp(jnp.minimum(h, 0.0)) - 1.0)
    h = h @ w2_ref[...] + bb2_ref[...]
    h = jnp.where(h > 0, h, jnp.exp(jnp.minimum(h, 0.0)) - 1.0)
    probs_ref[...] = h @ w3_ref[...] + bb3_ref[...]


@jax.jit
def _run(x_in, edge_index, edge_attr, params):
    p = params
    x = x_in[0]
    ea = edge_attr[0]
    src = edge_index[0, 0]
    dst = edge_index[0, 1]
    r = lambda b: b.reshape(1, -1)

    c1w1 = p['c1_W1']
    w1ds = jnp.concatenate([c1w1[:, :_D].T, c1w1[:, _D:2 * _D].T], axis=1)
    w1e = c1w1[:, 2 * _D:].T
    c2w1 = p['c2_W1']
    w2ds = jnp.concatenate([c2w1[:, :_OD // 2].T,
                            c2w1[:, _OD // 2:_OD].T], axis=0)
    w2e = c2w1[:, _OD:].T

    zeros128 = jnp.zeros((_NP, _OD), jnp.float32)

    pdps, skip, gate = pl.pallas_call(
        _node_pre_body,
        out_shape=(_f32((_N, _OD)), _f32((_N, _OD)), _f32((_N, _OD))),
        in_specs=[_full((_N, _D)), _full((1, _D)), _full((1, _D)),
                  _full((1, _D)), _full((_D, _OD)), _full((1, _OD)),
                  _full((_OD, _OD)), _full((1, _OD)),
                  _full((_D, _OD))],
        out_specs=(_full((_N, _OD)), _full((_N, _OD)), _full((_N, _OD))),
        grid=(1,),
    )(x, r(p['dummy']), r(p['bn0_g']), r(p['bn0_b']),
      p['proj_W'].T, r(p['proj_b']), p['gate_W'].T, r(p['gate_b']),
      w1ds)

    nblk = _E // _EB
    eenc = pl.pallas_call(
        _edge_enc_body,
        out_shape=_f32((_E, _OD)),
        in_specs=[_eblk(_ED),
                  _full((1, _ED)), _full((1, _ED)),
                  _full((_ED, _OD)), _full((1, _OD)),
                  _full((_OD, 2 * _OD)), _full((1, 2 * _OD)),
                  _full((2 * _OD, _OD)), _full((1, _OD)),
                  _full((_ED, _ED)), _full((1, _ED)),
                  _full((_ED, 1)), _full((1, 1))],
        out_specs=_eblk(_OD),
        grid=(nblk,),
    )(ea, r(p['ee_ln_g']), r(p['ee_ln_b']),
      p['ee_W1'].T, r(p['ee_b1']), p['ee_W2'].T, r(p['ee_b2']),
      p['ee_W3'].T, r(p['ee_b3']),
      p['ec_W1'].T, r(p['ec_b1']), p['ec_W2'].T, r(p['ec_b2']))

    g1 = _gather_pair(pdps, pdps, dst, src)

    m1 = pl.pallas_call(
        _conv1_body,
        out_shape=_f32((_E, _OD)),
        in_specs=[_eblk(_OD), _eblk(_OD),
                  _full((_OD, _OD // 2)), _full((1, _OD // 2)),
                  _full((_OD // 2, _OD // 2)), _full((1, _OD // 2)),
                  _full((_OD // 2, _OD // 2)), _full((1, _OD // 2))],
        out_specs=_eblk(_OD),
        grid=(nblk,),
    )(g1, eenc, w1e, r(p['c1_b1']),
      p['c1_W2'].T, r(p['c1_b2']), p['c1_W3'].T, r(p['c1_b3']))

    s1p = _scatter(m1, dst, zeros128)

    t2, cnt = pl.pallas_call(
        _node_mid_body,
        out_shape=(_f32((_N, _OD)), _f32((_N, 1))),
        in_specs=[_full((_NC * _NP, _OD)),
                  _full((1, _OD // 2)), _full((1, _OD // 2))],
        out_specs=(_full((_N, _OD)), _full((_N, 1))),
        grid=(1,),
    )(s1p, r(p['bn1_g']), r(p['bn1_b']))

    g2 = _gather_pair(t2, t2, dst, src)

    efsp = _scatter(eenc, dst, zeros128)

    m2 = pl.pallas_call(
        _conv2_body,
        out_shape=_f32((_E, _OD)),
        in_specs=[_eblk(_OD), _eblk(_OD),
                  _full((_OD, _OD)),
                  _full((_OD, _OD)), _full((1, _OD)),
                  _full((_OD, _OD)), _full((1, _OD)),
                  _full((_OD, _OD)), _full((1, _OD))],
        out_specs=_eblk(_OD),
        grid=(nblk,),
    )(eenc, g2, w2ds,
      w2e, r(p['c2_b1']),
      p['c2_W2'].T, r(p['c2_b2']), p['c2_W3'].T, r(p['c2_b3']))

    s2p = _scatter(m2, dst, zeros128)

    xfc, probs = pl.pallas_call(
        _node_fin_body,
        out_shape=(_f32((_N, 2 * _OD)), _f32((_N, 1))),
        in_specs=[_full((_NC * _NP, _OD)), _full((_N, 1)),
                  _full((_NC * _NP, _OD)),
                  _full((_N, _OD)), _full((_N, _OD)),
                  _full((1, _OD)), _full((1, _OD)),
                  _full((2 * _OD, _OD)), _full((1, _OD)),
                  _full((_OD, _OD // 2)), _full((1, _OD // 2)),
                  _full((_OD // 2, 1)), _full((1, 1))],
        out_specs=(_full((_N, 2 * _OD)), _full((_N, 1))),
        grid=(1,),
    )(s2p, cnt, efsp, skip, gate,
      r(p['bn2_g']), r(p['bn2_b']),
      p['np_W1'].T, r(p['np_b1']), p['np_W2'].T, r(p['np_b2']),
      p['np_W3'].T, r(p['np_b3']))

    return xfc, probs


def kernel(x_in, edge_index, edge_attr, params):
    return _run(x_in, edge_index, edge_attr, params)
